# Initial kernel scaffold; baseline (speedup 1.0000x reference)
import jax
import jax.numpy as jnp
from jax.experimental import pallas as pl


def kernel(inputs, embedding_table, linear_w, linear_b):
    raise NotImplementedError("write your pallas kernel here")


# speedup vs baseline: 49.0847x; 49.0847x over previous
import dataclasses
import functools

import jax
import jax.numpy as jnp
from jax import lax
from jax.experimental import pallas as pl
from jax.experimental.pallas import tpu as pltpu
from jax.experimental.pallas import tpu_sc as plsc

F = 26
D = 16
B = 4096
TD = F * D
NC = 2
NS = 16
NW = NC * NS
EPW = B // NW
CHUNK = 4
RPC = CHUNK * F
NCHUNK = EPW // CHUNK
GROUPS = EPW // D


_cp = pltpu.CompilerParams()
if "needs_layout_passes" in pltpu.CompilerParams.__dataclass_fields__:
    _cp = dataclasses.replace(_cp, needs_layout_passes=False)
if "use_tc_tiling_on_sc" in pltpu.CompilerParams.__dataclass_fields__:
    _cp = dataclasses.replace(_cp, use_tc_tiling_on_sc=False)


@functools.partial(
    pl.kernel,
    out_type=jax.ShapeDtypeStruct((B // D, D), jnp.float32),
    mesh=plsc.VectorSubcoreMesh(core_axis_name="c", subcore_axis_name="s"),
    compiler_params=_cp,
    scratch_types=[
        pltpu.VMEM((EPW * F,), jnp.int32),
        pltpu.VMEM((2, RPC, TD), jnp.float32),
        pltpu.VMEM((2, RPC, D), jnp.float32),
        pltpu.VMEM((D, EPW), jnp.float32),
        pltpu.VMEM((GROUPS, D), jnp.float32),
        pltpu.SemaphoreType.DMA((2,)),
        pltpu.SemaphoreType.DMA((2,)),
    ],
)
def _ffm_sc(tab_hbm, w_hbm, idx_hbm, out_hbm,
            idx_v, rows, wrows, acc_t, out_v, rsem, wsem):
    wid = lax.axis_index("s") * NC + lax.axis_index("c")
    pltpu.sync_copy(idx_hbm.at[pl.ds(wid * (EPW * F), EPW * F)], idx_v)

    def copies(c, b):
        off = c * RPC if isinstance(c, int) else pl.multiple_of(c * RPC, 8)
        isl = idx_v.at[pl.ds(off, RPC)]
        return (
            pltpu.make_async_copy(tab_hbm.at[isl], rows.at[b], rsem.at[b]),
            pltpu.make_async_copy(w_hbm.at[isl], wrows.at[b], wsem.at[b]),
        )

    for cp in copies(0, 0):
        cp.start()
    for cp in copies(1, 1):
        cp.start()

    @pl.loop(0, NCHUNK)
    def _chunk(c):
        b = lax.rem(c, 2)
        for cp in copies(c, b):
            cp.wait()

        @pl.loop(0, CHUNK)
        def _elem(e):
            base = e * F
            acc = jnp.zeros((D,), jnp.float32)
            for i in range(F):
                for j in range(i + 1, F):
                    acc = acc + (rows[b, base + i, pl.ds(j * D, D)]
                                 * rows[b, base + j, pl.ds(i * D, D)])
            for i in range(F):
                acc = acc + wrows[b, base + i, :]
            n = c * CHUNK + e
            plsc.store_scatter(
                acc_t,
                [lax.iota(jnp.int32, D), jnp.full((D,), n, jnp.int32)],
                acc,
            )

        @pl.when(c + 2 < NCHUNK)
        def _refill():
            for cp in copies(c + 2, b):
                cp.start()

    for g in range(GROUPS):
        r = jnp.zeros((D,), jnp.float32)
        for dd in range(D):
            r = r + acc_t[dd, pl.ds(g * D, D)]
        out_v[g, :] = r
    pltpu.sync_copy(out_v, out_hbm.at[pl.ds(wid * GROUPS, GROUPS)])


def kernel(inputs, embedding_table, linear_w, linear_b):
    rows_total, nf, d = embedding_table.shape
    offsets = (rows_total // nf) * jnp.arange(nf, dtype=jnp.int32)
    gidx = (inputs + offsets[None, :]).reshape(-1)
    table2d = embedding_table.reshape(rows_total, nf * d)
    w_pad = jnp.pad(linear_w, ((0, 0), (0, d - 1)))
    out = _ffm_sc(table2d, w_pad, gidx)
    return out.reshape(B, 1) + linear_b
